# baseline (device time: 26836 ns/iter reference)
import jax
import jax.numpy as jnp
from jax import lax
from jax.experimental import pallas as pl
from jax.experimental.pallas import tpu as pltpu

N_DEV = 16
N_CHUNKS = 8
D_PER_CHUNK = N_DEV // N_CHUNKS

_GELU_C = 0.7978845608028654


def _gelu(y):
    return 0.5 * y * (1.0 + jnp.tanh(_GELU_C * (y + 0.044715 * y * y * y)))


def kernel(x, w_mat):
    m_per, k = x.shape
    _, n = w_mat.shape
    n_per = n // N_DEV
    n_chunk = n // N_CHUNKS

    def body(x_ref, w_hbm, out_ref, w_vmem, y_buf,
             load_sems, send_sems, recv_sems):
        my_i = lax.axis_index("i")
        my_chunk = lax.div(my_i, D_PER_CHUNK)
        my_q = lax.rem(my_i, D_PER_CHUNK)
        my_rows = pl.ds(my_i * m_per, m_per)

        xb = x_ref[...].astype(jnp.bfloat16)

        def chunk_idx(t):
            return lax.rem(my_chunk + t, N_CHUNKS)

        def make_load(t):
            c = chunk_idx(t)
            return pltpu.make_async_copy(
                w_hbm.at[:, pl.ds(c * n_chunk, n_chunk)],
                w_vmem.at[t % 2],
                load_sems.at[t % 2],
            )

        make_load(0).start()

        sends = []
        for t in range(N_CHUNKS):
            c = chunk_idx(t)
            make_load(t).wait()
            if t + 1 < N_CHUNKS:
                make_load(t + 1).start()

            wb = w_vmem[t % 2].astype(jnp.bfloat16)
            yc = _gelu(
                jnp.dot(xb, wb, preferred_element_type=jnp.float32)
            ).astype(jnp.bfloat16)

            for q in range(D_PER_CHUNK):
                j = c * D_PER_CHUNK + q
                slot = t * D_PER_CHUNK + q
                y_buf[slot, :, :] = yc[:, q * n_per:(q + 1) * n_per]
                is_own = j == my_i

                @pl.when(is_own)
                def _():
                    out_ref[my_rows, :] = y_buf[slot, :, :]

                rdma = pltpu.make_async_remote_copy(
                    src_ref=y_buf.at[slot],
                    dst_ref=out_ref.at[my_rows, :],
                    send_sem=send_sems.at[slot],
                    recv_sem=recv_sems.at[my_i],
                    device_id=(j,),
                    device_id_type=pl.DeviceIdType.MESH,
                )

                @pl.when(jnp.logical_not(is_own))
                def _():
                    rdma.start()

                sends.append((rdma, is_own))

        for s in range(N_DEV):
            recv = pltpu.make_async_remote_copy(
                src_ref=y_buf.at[0],
                dst_ref=out_ref.at[pl.ds(s * m_per, m_per), :],
                send_sem=send_sems.at[0],
                recv_sem=recv_sems.at[s],
                device_id=(s,),
                device_id_type=pl.DeviceIdType.MESH,
            )

            @pl.when(s != my_i)
            def _():
                recv.wait_recv()

        for rdma, is_own in sends:
            @pl.when(jnp.logical_not(is_own))
            def _():
                rdma.wait_send()

    return pl.pallas_call(
        body,
        out_shape=jax.ShapeDtypeStruct((N_DEV * m_per, n_per), jnp.bfloat16),
        in_specs=[
            pl.BlockSpec(memory_space=pltpu.VMEM),
            pl.BlockSpec(memory_space=pl.ANY),
        ],
        out_specs=pl.BlockSpec(memory_space=pltpu.VMEM),
        scratch_shapes=[
            pltpu.VMEM((2, k, n_chunk), jnp.float32),
            pltpu.VMEM((N_DEV, m_per, n_per), jnp.bfloat16),
            pltpu.SemaphoreType.DMA((2,)),
            pltpu.SemaphoreType.DMA((N_DEV,)),
            pltpu.SemaphoreType.DMA((N_DEV,)),
        ],
    )(x, w_mat)
